# baseline (speedup 1.0000x reference)
import functools

import jax
import jax.numpy as jnp
from jax import lax
from jax.experimental import pallas as pl
from jax.experimental.pallas import tpu as pltpu
from jax.experimental.pallas import tpu_sc as plsc

D = 64
SEQ = 200
BATCH = 4096
NTOK = BATCH * SEQ
VROWS = 1000000 * D // 128

NC = 2
NS = 16
NW = NC * NS
B_PER_W = BATCH // NW
TOK_PER_W = B_PER_W * SEQ
NG = B_PER_W // 16


def _rsqrt_vec(v):
    i = plsc.bitcast(v, jnp.int32)
    y = plsc.bitcast(jnp.full((16,), 0x5F3759DF, jnp.int32) - (i >> 1),
                     jnp.float32)
    y = y * (1.5 - 0.5 * v * y * y)
    y = y * (1.5 - 0.5 * v * y * y)
    return y


def _make_sc_kernel():
    mesh = plsc.VectorSubcoreMesh(core_axis_name="c", subcore_axis_name="s")

    @functools.partial(
        pl.kernel,
        mesh=mesh,
        compiler_params=pltpu.CompilerParams(
            needs_layout_passes=False, use_tc_tiling_on_sc=True),
        out_type=jax.ShapeDtypeStruct((SEQ, D, BATCH), jnp.float32),
        scratch_types=[
            pltpu.VMEM((TOK_PER_W,), jnp.int32),
            pltpu.VMEM((2, B_PER_W), jnp.int32),
            pltpu.VMEM((2, B_PER_W), jnp.int32),
            pltpu.VMEM((2, B_PER_W, 128), jnp.float32),
            pltpu.VMEM((2, D, B_PER_W), jnp.float32),
            pltpu.VMEM((2, 16 * D), jnp.float32),
            pltpu.VMEM((16 * D,), jnp.float32),
            pltpu.VMEM((16 * D,), jnp.float32),
            pltpu.SemaphoreType.DMA,
            pltpu.SemaphoreType.DMA,
            pltpu.SemaphoreType.DMA,
            pltpu.SemaphoreType.DMA,
        ],
    )
    def emb_kernel(xf_hbm, tok2_hbm, posb_hbm, gb_hbm, bb_hbm, out_hbm,
                   idx_all, gidx_v, colb_v, rows_v, obuf_v, posr_v,
                   gb_v, bb_v, gsem0, gsem1, osem0, osem1):
        gsem = [gsem0, gsem1]
        osem = [osem0, osem1]
        wid = lax.axis_index("s") * NC + lax.axis_index("c")
        base0 = pl.multiple_of(wid * TOK_PER_W, 8)
        pltpu.sync_copy(xf_hbm.at[pl.ds(base0, TOK_PER_W)], idx_all)
        pltpu.sync_copy(gb_hbm, gb_v)
        pltpu.sync_copy(bb_hbm, bb_v)

        iota = jnp.arange(16, dtype=jnp.int32)
        bcol0 = pl.multiple_of(wid * B_PER_W, 8)

        def build_lists(s, b):
            for j in range(NG):
                iv = (iota + (16 * j)) * SEQ + s
                tv = plsc.load_gather(idx_all, [iv])
                gidx_v[b, pl.ds(16 * j, 16)] = tv >> 1
                colb_v[b, pl.ds(16 * j, 16)] = (tv & 1) << 6

        def fire(s, b):
            pltpu.async_copy(tok2_hbm.at[gidx_v.at[b]], rows_v.at[b],
                             gsem[b])
            pltpu.async_copy(posb_hbm.at[s], posr_v.at[b], gsem[b])

        def wait_gather(b):
            pltpu.make_async_copy(tok2_hbm.at[gidx_v.at[b]],
                                  rows_v.at[b], gsem[b]).wait()
            pltpu.make_async_copy(posb_hbm.at[0], posr_v.at[b],
                                  gsem[b]).wait()

        def fire_out(s, b):
            pltpu.async_copy(
                obuf_v.at[b],
                out_hbm.at[s, :, pl.ds(bcol0, B_PER_W)], osem[b])

        def wait_out(b):
            pltpu.make_async_copy(
                obuf_v.at[b],
                out_hbm.at[0, :, pl.ds(bcol0, B_PER_W)], osem[b]).wait()

        def compute(b):
            @plsc.parallel_loop(0, NG, 1)
            def group_body(g):
                colb = colb_v[b, pl.ds(16 * g, 16)]
                rowv = iota + 16 * g
                sa = [jnp.zeros((16,), jnp.float32) for _ in range(4)]
                qa = [jnp.zeros((16,), jnp.float32) for _ in range(4)]
                for f in range(D):
                    hv = plsc.load_gather(rows_v.at[b], [rowv, colb + f]) \
                        + posr_v[b, pl.ds(16 * f, 16)]
                    obuf_v[b, f, pl.ds(16 * g, 16)] = hv
                    sa[f & 3] = sa[f & 3] + hv
                    qa[f & 3] = qa[f & 3] + hv * hv
                stot = (sa[0] + sa[1]) + (sa[2] + sa[3])
                qtot = (qa[0] + qa[1]) + (qa[2] + qa[3])
                mean = stot * (1.0 / D)
                var = qtot * (1.0 / D) - mean * mean
                rstd = _rsqrt_vec(var + 1e-5)
                for f in range(D):
                    hv = obuf_v[b, f, pl.ds(16 * g, 16)]
                    rg = rstd * gb_v[pl.ds(16 * f, 16)]
                    obuf_v[b, f, pl.ds(16 * g, 16)] = (
                        (hv - mean) * rg + bb_v[pl.ds(16 * f, 16)])

        build_lists(0, 0)
        fire(0, 0)

        def pair_body(p, carry):
            for bb in range(2):
                s = 2 * p + bb
                wait_gather(bb)

                @pl.when(s < SEQ - 1)
                def _():
                    build_lists(s + 1, 1 - bb)
                    fire(s + 1, 1 - bb)

                @pl.when(s > 1)
                def _():
                    wait_out(bb)

                compute(bb)
                fire_out(s, bb)
            return carry

        lax.fori_loop(0, SEQ // 2, pair_body, 0)
        wait_out(0)
        wait_out(1)

    return emb_kernel


_emb_kernel = _make_sc_kernel()


@jax.jit
def kernel(x, tok_embed, pos_embed, gamma, beta):
    xf = x.reshape(-1).astype(jnp.int32)
    tok2 = tok_embed.reshape(VROWS, 128)
    posb = jnp.broadcast_to(pos_embed[:, :, None], (SEQ, D, 16)).reshape(
        SEQ, 16 * D)
    gb = jnp.broadcast_to(gamma[:, None], (D, 16)).reshape(16 * D)
    bb = jnp.broadcast_to(beta[:, None], (D, 16)).reshape(16 * D)
    z = _emb_kernel(xf, tok2, posb, gb, bb)
    return jnp.transpose(z, (2, 0, 1))


# speedup vs baseline: 1.3549x; 1.3549x over previous
import functools

import jax
import jax.numpy as jnp
from jax import lax
from jax.experimental import pallas as pl
from jax.experimental.pallas import tpu as pltpu
from jax.experimental.pallas import tpu_sc as plsc

D = 64
SEQ = 200
BATCH = 4096
NTOK = BATCH * SEQ
VROWS = 1000000 * D // 128

NC = 2
NS = 16
NW = NC * NS
B_PER_W = BATCH // NW
TOK_PER_W = B_PER_W * SEQ
NG = B_PER_W // 16


def _rsqrt_vec(v):
    i = plsc.bitcast(v, jnp.int32)
    y = plsc.bitcast(jnp.full((16,), 0x5F3759DF, jnp.int32) - (i >> 1),
                     jnp.float32)
    y = y * (1.5 - 0.5 * v * y * y)
    y = y * (1.5 - 0.5 * v * y * y)
    return y


def _make_sc_kernel():
    mesh = plsc.VectorSubcoreMesh(core_axis_name="c", subcore_axis_name="s")

    @functools.partial(
        pl.kernel,
        mesh=mesh,
        compiler_params=pltpu.CompilerParams(
            needs_layout_passes=False, use_tc_tiling_on_sc=True),
        out_type=jax.ShapeDtypeStruct((SEQ, D, BATCH), jnp.float32),
        scratch_types=[
            pltpu.VMEM((TOK_PER_W,), jnp.int32),
            pltpu.VMEM((B_PER_W,), jnp.int32),
            pltpu.VMEM((B_PER_W,), jnp.int32),
            pltpu.VMEM((B_PER_W,), jnp.int32),
            pltpu.VMEM((B_PER_W,), jnp.int32),
            pltpu.VMEM((B_PER_W, 128), jnp.float32),
            pltpu.VMEM((B_PER_W, 128), jnp.float32),
            pltpu.VMEM((D, B_PER_W), jnp.float32),
            pltpu.VMEM((D, B_PER_W), jnp.float32),
            pltpu.VMEM((16 * D,), jnp.float32),
            pltpu.VMEM((16 * D,), jnp.float32),
            pltpu.VMEM((16 * D,), jnp.float32),
            pltpu.VMEM((16 * D,), jnp.float32),
            pltpu.VMEM((16 * D,), jnp.float32),
            pltpu.SemaphoreType.DMA,
            pltpu.SemaphoreType.DMA,
            pltpu.SemaphoreType.DMA,
            pltpu.SemaphoreType.DMA,
        ],
    )
    def emb_kernel(xf_hbm, tok2_hbm, posb_hbm, gb_hbm, bb_hbm, out_hbm,
                   idx_all, gidx0, gidx1, colb0, colb1, rows0, rows1,
                   obuf0, obuf1, posr0, posr1, posrot_v, gb_v, bb_v,
                   gsem0, gsem1, osem0, osem1):
        gidx = [gidx0, gidx1]
        colb = [colb0, colb1]
        rows = [rows0, rows1]
        obuf = [obuf0, obuf1]
        posr = [posr0, posr1]
        gsem = [gsem0, gsem1]
        osem = [osem0, osem1]
        wid = lax.axis_index("s") * NC + lax.axis_index("c")
        base0 = pl.multiple_of(wid * TOK_PER_W, 8)
        pltpu.sync_copy(xf_hbm.at[pl.ds(base0, TOK_PER_W)], idx_all)
        pltpu.sync_copy(gb_hbm, gb_v)
        pltpu.sync_copy(bb_hbm, bb_v)

        iota = jnp.arange(16, dtype=jnp.int32)
        bcol0 = pl.multiple_of(wid * B_PER_W, 8)

        def build_lists(s, b):
            for j in range(NG):
                iv = (iota + (16 * j)) * SEQ + s
                tv = plsc.load_gather(idx_all, [iv])
                gidx[b][pl.ds(16 * j, 16)] = tv >> 1
                colb[b][pl.ds(16 * j, 16)] = (tv & 1) << 6

        def fire(s, b):
            pltpu.async_copy(tok2_hbm.at[gidx[b]], rows[b], gsem[b])
            pltpu.async_copy(posb_hbm.at[s], posr[b], gsem[b])

        def wait_gather(b):
            pltpu.make_async_copy(tok2_hbm.at[gidx[b]], rows[b],
                                  gsem[b]).wait()
            pltpu.make_async_copy(posb_hbm.at[0], posr[b], gsem[b]).wait()

        def fire_out(s, b):
            pltpu.async_copy(
                obuf[b], out_hbm.at[s, :, pl.ds(bcol0, B_PER_W)], osem[b])

        def wait_out(b):
            pltpu.make_async_copy(
                obuf[b], out_hbm.at[0, :, pl.ds(bcol0, B_PER_W)],
                osem[b]).wait()

        def compute(b):
            qv = iota
            for f in range(D):
                posrot_v[pl.ds(16 * f, 16)] = plsc.load_gather(
                    posr[b], [(qv << 4) + iota])
                qv = (qv + 1) & 63

            @plsc.parallel_loop(0, NG, 1)
            def group_body(g):
                cb = colb[b][pl.ds(16 * g, 16)]
                rowv = iota + 16 * g
                sa = [jnp.zeros((16,), jnp.float32) for _ in range(4)]
                qa = [jnp.zeros((16,), jnp.float32) for _ in range(4)]
                qv2 = iota
                for f in range(D):
                    hv = plsc.load_gather(rows[b], [rowv, cb + qv2]) \
                        + posrot_v[pl.ds(16 * f, 16)]
                    plsc.store_scatter(obuf[b], [qv2, rowv], hv)
                    sa[f & 3] = sa[f & 3] + hv
                    qa[f & 3] = qa[f & 3] + hv * hv
                    qv2 = (qv2 + 1) & 63
                stot = (sa[0] + sa[1]) + (sa[2] + sa[3])
                qtot = (qa[0] + qa[1]) + (qa[2] + qa[3])
                mean = stot * (1.0 / D)
                var = qtot * (1.0 / D) - mean * mean
                rstd = _rsqrt_vec(var + 1e-5)
                for f in range(D):
                    hv = obuf[b][f, pl.ds(16 * g, 16)]
                    rg = rstd * gb_v[pl.ds(16 * f, 16)]
                    obuf[b][f, pl.ds(16 * g, 16)] = (
                        (hv - mean) * rg + bb_v[pl.ds(16 * f, 16)])

        build_lists(0, 0)
        fire(0, 0)

        def pair_body(p, carry):
            for bb in range(2):
                s = 2 * p + bb
                wait_gather(bb)

                @pl.when(s < SEQ - 1)
                def _():
                    build_lists(s + 1, 1 - bb)
                    fire(s + 1, 1 - bb)

                @pl.when(s > 1)
                def _():
                    wait_out(bb)

                compute(bb)
                fire_out(s, bb)
            return carry

        lax.fori_loop(0, SEQ // 2, pair_body, 0)
        wait_out(0)
        wait_out(1)

    return emb_kernel


_emb_kernel = _make_sc_kernel()


@jax.jit
def kernel(x, tok_embed, pos_embed, gamma, beta):
    xf = x.reshape(-1).astype(jnp.int32)
    tok2 = tok_embed.reshape(VROWS, 128)
    posb = jnp.broadcast_to(pos_embed[:, :, None], (SEQ, D, 16)).reshape(
        SEQ, 16 * D)
    gb = jnp.broadcast_to(gamma[:, None], (D, 16)).reshape(16 * D)
    bb = jnp.broadcast_to(beta[:, None], (D, 16)).reshape(16 * D)
    z = _emb_kernel(xf, tok2, posb, gb, bb)
    return jnp.transpose(z, (2, 0, 1))
